# baseline (speedup 1.0000x reference)
import functools

import jax
import jax.numpy as jnp
from jax import lax
from jax.experimental import pallas as pl
from jax.experimental.pallas import tpu as pltpu
from jax.experimental.pallas import tpu_sc as plsc

DP_CONST = 0.2
RATIO = 1.0 / (1.0 - DP_CONST)
N_EDGES_CONST = 6400000
NUM_WORKERS = 32
PER_WORKER = N_EDGES_CONST // NUM_WORKERS
CHUNK = 20000
N_CHUNKS = PER_WORKER // CHUNK
LANES = 16


def _drop_adj_sc(edge_values, rand_vals):
    mesh = plsc.VectorSubcoreMesh(core_axis_name="c", subcore_axis_name="s")

    @functools.partial(
        pl.kernel,
        mesh=mesh,
        out_type=jax.ShapeDtypeStruct((N_EDGES_CONST,), jnp.float32),
        scratch_types=[
            pltpu.VMEM((CHUNK,), jnp.float32),
            pltpu.VMEM((CHUNK,), jnp.float32),
        ],
    )
    def k(vals_hbm, rand_hbm, out_hbm, vals_v, rand_v):
        wid = lax.axis_index("s") * 2 + lax.axis_index("c")
        base = wid * PER_WORKER

        def chunk_body(ci, carry):
            off = base + ci * CHUNK
            pltpu.sync_copy(vals_hbm.at[pl.ds(off, CHUNK)], vals_v)
            pltpu.sync_copy(rand_hbm.at[pl.ds(off, CHUNK)], rand_v)

            def body(i, c):
                sl = pl.ds(i * LANES, LANES)
                v = vals_v[sl]
                r = rand_v[sl]
                vals_v[sl] = jnp.where(r > DP_CONST, v * RATIO,
                                       jnp.float32(0.0))
                return c

            lax.fori_loop(0, CHUNK // LANES, body, 0)
            pltpu.sync_copy(vals_v, out_hbm.at[pl.ds(off, CHUNK)])
            return carry

        lax.fori_loop(0, N_CHUNKS, chunk_body, 0)

    return k(edge_values, rand_vals)


def kernel(edge_index, edge_values, rand_vals):
    return edge_index, _drop_adj_sc(edge_values, rand_vals)


# speedup vs baseline: 1.2456x; 1.2456x over previous
import functools

import jax
import jax.numpy as jnp
from jax import lax
from jax.experimental import pallas as pl
from jax.experimental.pallas import tpu as pltpu
from jax.experimental.pallas import tpu_sc as plsc

DP_CONST = 0.2
RATIO = 1.0 / (1.0 - DP_CONST)
N_EDGES_CONST = 6400000
NUM_WORKERS = 32
PER_WORKER = N_EDGES_CONST // NUM_WORKERS
CHUNK = 10000
N_CHUNKS = PER_WORKER // CHUNK
LANES = 16


def _drop_adj_sc(edge_values, rand_vals):
    mesh = plsc.VectorSubcoreMesh(core_axis_name="c", subcore_axis_name="s")

    @functools.partial(
        pl.kernel,
        mesh=mesh,
        out_type=jax.ShapeDtypeStruct((N_EDGES_CONST,), jnp.float32),
        scratch_types=[
            pltpu.VMEM((CHUNK,), jnp.float32),
            pltpu.VMEM((CHUNK,), jnp.float32),
            pltpu.VMEM((CHUNK,), jnp.float32),
            pltpu.VMEM((CHUNK,), jnp.float32),
            pltpu.VMEM((CHUNK,), jnp.float32),
            pltpu.VMEM((CHUNK,), jnp.float32),
            pltpu.SemaphoreType.DMA,
            pltpu.SemaphoreType.DMA,
            pltpu.SemaphoreType.DMA,
            pltpu.SemaphoreType.DMA,
            pltpu.SemaphoreType.DMA,
            pltpu.SemaphoreType.DMA,
        ],
    )
    def k(vals_hbm, rand_hbm, out_hbm,
          iv0, ir0, ob0, iv1, ir1, ob1,
          sv0, sr0, so0, sv1, sr1, so1):
        wid = lax.axis_index("s") * 2 + lax.axis_index("c")
        base = wid * PER_WORKER
        bufs = ((iv0, ir0, ob0), (iv1, ir1, ob1))
        svs = (sv0, sv1)
        srs = (sr0, sr1)
        sos = (so0, so1)
        hv = [None, None]
        hr = [None, None]
        ho = [None, None]

        def start_in(ci):
            b = ci % 2
            off = base + ci * CHUNK
            hv[b] = pltpu.async_copy(
                vals_hbm.at[pl.ds(off, CHUNK)], bufs[b][0], svs[b])
            hr[b] = pltpu.async_copy(
                rand_hbm.at[pl.ds(off, CHUNK)], bufs[b][1], srs[b])

        start_in(0)
        start_in(1)
        for ci in range(N_CHUNKS):
            b = ci % 2
            hv[b].wait()
            hr[b].wait()
            if ci >= 2:
                ho[b].wait()
            iv, ir, ob = bufs[b]

            def body(i, c, iv=iv, ir=ir, ob=ob):
                sl = pl.ds(i * LANES, LANES)
                v = iv[sl]
                r = ir[sl]
                ob[sl] = jnp.where(r > DP_CONST, v * RATIO, jnp.float32(0.0))
                return c

            lax.fori_loop(0, CHUNK // LANES, body, 0, unroll=8)
            off = base + ci * CHUNK
            ho[b] = pltpu.async_copy(ob, out_hbm.at[pl.ds(off, CHUNK)], sos[b])
            if ci + 2 < N_CHUNKS:
                start_in(ci + 2)
        ho[0].wait()
        ho[1].wait()

    return k(edge_values, rand_vals)


def kernel(edge_index, edge_values, rand_vals):
    return edge_index, _drop_adj_sc(edge_values, rand_vals)


# speedup vs baseline: 1.8937x; 1.5204x over previous
import functools

import jax
import jax.numpy as jnp
from jax import lax
from jax.experimental import pallas as pl
from jax.experimental.pallas import tpu as pltpu
from jax.experimental.pallas import tpu_sc as plsc

DP_CONST = 0.2
RATIO = 1.0 / (1.0 - DP_CONST)
N_EDGES_CONST = 6400000
NUM_WORKERS = 32
PER_WORKER = N_EDGES_CONST // NUM_WORKERS
CHUNK = 10000
N_CHUNKS = PER_WORKER // CHUNK
LANES = 16
NBUF_IN = 4
NBUF_OUT = 2


def _drop_adj_sc(edge_values, rand_vals):
    mesh = plsc.VectorSubcoreMesh(core_axis_name="c", subcore_axis_name="s")

    vmem = lambda: pltpu.VMEM((CHUNK,), jnp.float32)

    @functools.partial(
        pl.kernel,
        mesh=mesh,
        out_type=jax.ShapeDtypeStruct((N_EDGES_CONST,), jnp.float32),
        scratch_types=(
            [vmem() for _ in range(2 * NBUF_IN + NBUF_OUT)]
            + [pltpu.SemaphoreType.DMA] * (2 * NBUF_IN + NBUF_OUT)
        ),
    )
    def k(vals_hbm, rand_hbm, out_hbm, *scratch):
        ivs = scratch[0:NBUF_IN]
        irs = scratch[NBUF_IN:2 * NBUF_IN]
        obs = scratch[2 * NBUF_IN:2 * NBUF_IN + NBUF_OUT]
        sems = scratch[2 * NBUF_IN + NBUF_OUT:]
        svs = sems[0:NBUF_IN]
        srs = sems[NBUF_IN:2 * NBUF_IN]
        sos = sems[2 * NBUF_IN:]

        wid = lax.axis_index("s") * 2 + lax.axis_index("c")
        base = wid * PER_WORKER

        hv = [None] * NBUF_IN
        hr = [None] * NBUF_IN
        ho = [None] * NBUF_OUT

        def start_in(ci):
            b = ci % NBUF_IN
            off = base + ci * CHUNK
            hv[b] = pltpu.async_copy(
                vals_hbm.at[pl.ds(off, CHUNK)], ivs[b], svs[b])
            hr[b] = pltpu.async_copy(
                rand_hbm.at[pl.ds(off, CHUNK)], irs[b], srs[b])

        for ci in range(NBUF_IN):
            start_in(ci)

        for ci in range(N_CHUNKS):
            b = ci % NBUF_IN
            ob_b = ci % NBUF_OUT
            hv[b].wait()
            hr[b].wait()
            if ci >= NBUF_OUT:
                ho[ob_b].wait()
            iv, ir, ob = ivs[b], irs[b], obs[ob_b]

            @plsc.parallel_loop(0, CHUNK, step=LANES, unroll=8)
            def body(i, iv=iv, ir=ir, ob=ob):
                sl = pl.ds(i, LANES)
                v = iv[sl]
                r = ir[sl]
                ob[sl] = jnp.where(r > DP_CONST, v * RATIO, jnp.float32(0.0))

            off = base + ci * CHUNK
            ho[ob_b] = pltpu.async_copy(
                ob, out_hbm.at[pl.ds(off, CHUNK)], sos[ob_b])
            if ci + NBUF_IN < N_CHUNKS:
                start_in(ci + NBUF_IN)
        for b in range(NBUF_OUT):
            ho[b].wait()

    return k(edge_values, rand_vals)


def kernel(edge_index, edge_values, rand_vals):
    return edge_index, _drop_adj_sc(edge_values, rand_vals)
